# Initial kernel scaffold; baseline (speedup 1.0000x reference)
import jax
import jax.numpy as jnp
from jax.experimental import pallas as pl


def kernel(inputs, class_bias, global_bias):
    raise NotImplementedError("write your pallas kernel here")


# speedup vs baseline: 2.3141x; 2.3141x over previous
import functools
import math

import jax
import jax.numpy as jnp
from jax import lax
from jax.experimental import pallas as pl
from jax.experimental.pallas import tpu as pltpu
from jax.experimental.pallas import tpu_sc as plsc


def _make_kernel(B, F, V, n_workers, n_cores, P):
    N = B * F
    n_per_w = N // n_workers
    n_vreg = n_per_w // 16
    p_vreg = P // 16

    mesh = plsc.VectorSubcoreMesh(core_axis_name="c", subcore_axis_name="s")

    @functools.partial(
        pl.kernel,
        out_type=jax.ShapeDtypeStruct((N,), jnp.float32),
        mesh=mesh,
        scratch_types=[
            pltpu.VMEM((n_per_w,), jnp.int32),
            pltpu.VMEM((n_per_w,), jnp.float32),
            pltpu.VMEM((P,), jnp.int32),
            pltpu.VMEM((P,), jnp.float32),
            pltpu.SemaphoreType.DMA,
        ],
    )
    def k(idx_hbm, off_hbm, gbt_hbm, cb_hbm, out_hbm,
          idx_v, val_v, off_v, gbt_v, sem):
        wid = lax.axis_index("s") * n_cores + lax.axis_index("c")
        base = wid * n_per_w
        pltpu.sync_copy(off_hbm, off_v)
        pltpu.sync_copy(gbt_hbm, gbt_v)
        pltpu.sync_copy(idx_hbm.at[pl.ds(base, n_per_w)], idx_v)

        def to_flat(j, _):
            s = lax.rem(j, p_vreg) * 16
            idx_v[pl.ds(j * 16, 16)] = (
                idx_v[pl.ds(j * 16, 16)] + off_v[pl.ds(s, 16)])
            return 0

        lax.fori_loop(0, n_vreg, to_flat, 0)

        pltpu.async_copy(cb_hbm.at[idx_v], val_v, sem).wait()

        def add_gb(j, _):
            s = lax.rem(j, p_vreg) * 16
            val_v[pl.ds(j * 16, 16)] = (
                val_v[pl.ds(j * 16, 16)] + gbt_v[pl.ds(s, 16)])
            return 0

        lax.fori_loop(0, n_vreg, add_gb, 0)
        pltpu.sync_copy(val_v, out_hbm.at[pl.ds(base, n_per_w)])

    return k


def kernel(inputs, class_bias, global_bias):
    B, F = inputs.shape
    _, V = class_bias.shape
    info = plsc.get_sparse_core_info()
    n_workers = info.num_cores * info.num_subcores
    P = 16 * F // math.gcd(16, F)
    idx_flat = inputs.astype(jnp.int32).reshape(B * F)
    cb_flat = class_bias.reshape(F * V)
    fpat = jnp.arange(P, dtype=jnp.int32) % F
    off_tile = fpat * V
    gb_tile = global_bias[fpat]
    k = _make_kernel(B, F, V, n_workers, info.num_cores, P)
    out_flat = k(idx_flat, off_tile, gb_tile, cb_flat)
    return out_flat.reshape(B, F)


# speedup vs baseline: 4.4201x; 1.9101x over previous
import functools
import math

import jax
import jax.numpy as jnp
from jax import lax
from jax.experimental import pallas as pl
from jax.experimental.pallas import tpu as pltpu
from jax.experimental.pallas import tpu_sc as plsc


def _make_kernel(B, F, V, n_workers, n_cores):
    N = B * F
    n_per_w = N // n_workers
    n_vreg = n_per_w // 16
    g = math.gcd(B, n_per_w) // 16
    unroll = math.gcd(8, g if g > 0 else 1)

    mesh = plsc.VectorSubcoreMesh(core_axis_name="c", subcore_axis_name="s")

    @functools.partial(
        pl.kernel,
        out_type=jax.ShapeDtypeStruct((N,), jnp.float32),
        mesh=mesh,
        scratch_types=[
            pltpu.VMEM((n_per_w,), jnp.int32),
            pltpu.VMEM((n_per_w,), jnp.float32),
            pltpu.VMEM((16,), jnp.float32),
            pltpu.VMEM((16,), jnp.float32),
            pltpu.SemaphoreType.DMA,
            pltpu.SemaphoreType.DMA,
        ],
    )
    def k(idx_hbm, cb_hbm, gb_hbm, out_hbm, idx_v, val_v, gb0_v, gb1_v,
          sem, semg):
        wid = lax.axis_index("s") * n_cores + lax.axis_index("c")
        base = wid * n_per_w
        f0 = lax.div(base, B)
        f1 = lax.div(base + (n_per_w - 1), B)
        jb = lax.min((f0 + 1) * B - base, n_per_w) // 16
        jb_blk = jb // unroll
        n_blk = n_vreg // unroll

        cp_idx = pltpu.make_async_copy(
            idx_hbm.at[pl.ds(base, n_per_w)], idx_v, sem)
        cp_idx.start()
        pltpu.async_copy(
            gb_hbm.at[jnp.full((16,), f0, jnp.int32)], gb0_v, semg).wait()
        pltpu.async_copy(
            gb_hbm.at[jnp.full((16,), f1, jnp.int32)], gb1_v, semg).wait()
        gb0 = gb0_v[...]
        gb1 = gb1_v[...]
        cp_idx.wait()

        def add_off(off):
            def body(j, _):
                for u in range(unroll):
                    s = (j * unroll + u) * 16
                    idx_v[pl.ds(s, 16)] = idx_v[pl.ds(s, 16)] + off
                return 0
            return body

        off0 = jnp.full((16,), f0 * V, jnp.int32)
        off1 = jnp.full((16,), f1 * V, jnp.int32)
        lax.fori_loop(0, jb_blk, add_off(off0), 0)
        lax.fori_loop(jb_blk, n_blk, add_off(off1), 0)

        pltpu.async_copy(cb_hbm.at[idx_v], val_v, sem).wait()

        def add_gb(gbv):
            def body(j, _):
                for u in range(unroll):
                    s = (j * unroll + u) * 16
                    val_v[pl.ds(s, 16)] = val_v[pl.ds(s, 16)] + gbv
                return 0
            return body

        lax.fori_loop(0, jb_blk, add_gb(gb0), 0)
        lax.fori_loop(jb_blk, n_blk, add_gb(gb1), 0)
        pltpu.sync_copy(val_v, out_hbm.at[pl.ds(base, n_per_w)])

    return k


def kernel(inputs, class_bias, global_bias):
    B, F = inputs.shape
    _, V = class_bias.shape
    info = plsc.get_sparse_core_info()
    n_workers = info.num_cores * info.num_subcores
    idx_flat = inputs.astype(jnp.int32).T.reshape(F * B)
    cb_flat = class_bias.reshape(F * V)
    k = _make_kernel(B, F, V, n_workers, info.num_cores)
    out_flat = k(idx_flat, cb_flat, global_bias)
    return out_flat.reshape(F, B).T


# speedup vs baseline: 7.3801x; 1.6697x over previous
import functools
import jax
import jax.numpy as jnp
from jax import lax
from jax.experimental import pallas as pl
from jax.experimental.pallas import tpu as pltpu
from jax.experimental.pallas import tpu_sc as plsc


def kernel(inputs, class_bias, global_bias):
    B, F = inputs.shape
    _, V = class_bias.shape
    info = plsc.get_sparse_core_info()
    NC = info.num_cores
    mesh = plsc.VectorSubcoreMesh(core_axis_name="c", subcore_axis_name="s")

    @functools.partial(
        pl.kernel,
        out_type=jax.ShapeDtypeStruct((F, B), jnp.int32),
        mesh=mesh,
        compiler_params=pltpu.CompilerParams(needs_layout_passes=False),
        scratch_types=[
            pltpu.VMEM((V,), jnp.float32),
            pltpu.VMEM((B,), jnp.int32),
            pltpu.VMEM((16,), jnp.float32),
            pltpu.SemaphoreType.DMA,
        ],
    )
    def k(idx_hbm, cb_hbm, gb_hbm, out_hbm, tab_v, idx_v, gbs_v, sem):
        wid = lax.axis_index("s") * NC + lax.axis_index("c")

        @pl.when(wid < F)
        def _():
            pltpu.sync_copy(cb_hbm.at[wid], tab_v)
            pltpu.sync_copy(idx_hbm.at[wid], idx_v)
            pltpu.async_copy(
                gb_hbm.at[jnp.full((16,), wid, jnp.int32)], gbs_v, sem).wait()
            gbv = gbs_v[...]

            def body(j, _):
                for u in range(8):
                    s = (j * 8 + u) * 16
                    iv = idx_v[pl.ds(s, 16)]
                    res = plsc.load_gather(tab_v, [iv]) + gbv
                    idx_v[pl.ds(s, 16)] = plsc.bitcast(res, jnp.int32)
                return 0

            lax.fori_loop(0, B // 128, body, 0)
            pltpu.sync_copy(idx_v, out_hbm.at[wid])

    idx_t = inputs.astype(jnp.int32).T
    out_t = lax.bitcast_convert_type(k(idx_t, class_bias, global_bias),
                                     jnp.float32)
    return out_t.T
